# baseline (speedup 1.0000x reference)
import functools

import jax
import jax.numpy as jnp
from jax import lax
from jax.experimental import pallas as pl
from jax.experimental.pallas import tpu as pltpu
from jax.experimental.pallas import tpu_sc as plsc

_B = 4
_N = 2048
_K = 10

_TCR = 768
_R = 768
_JT = _TCR // _R

_NC = 2
_NS = 16
_NW = _NC * _NS
_CPB = _NW // _B
_SCR = _N - _TCR
_RPW = _SCR // _CPB
_IL = 4
_NCHUNK = _N // 16


def _sqrt16(x):
    i = lax.bitcast_convert_type(x, jnp.int32)
    y = lax.bitcast_convert_type(jnp.int32(0x5F3759DF) - (i >> 1), jnp.float32)
    for _ in range(3):
        y = y * (jnp.float32(1.5) - jnp.float32(0.5) * x * y * y)
    return x * y


_mesh = plsc.VectorSubcoreMesh(
    core_axis_name="c", subcore_axis_name="s", num_cores=_NC, num_subcores=_NS
)


@functools.partial(
    pl.kernel,
    out_type=jax.ShapeDtypeStruct((_NW, 16), jnp.float32),
    mesh=_mesh,
    compiler_params=pltpu.CompilerParams(needs_layout_passes=False),
    scratch_types=[
        pltpu.VMEM((_N,), jnp.float32),
        pltpu.VMEM((_N,), jnp.float32),
        pltpu.VMEM((_N,), jnp.float32),
        pltpu.VMEM((_N,), jnp.float32),
        pltpu.VMEM((_N,), jnp.float32),
        pltpu.VMEM((16,), jnp.float32),
    ],
)
def _sc_topk(x_hbm, y_hbm, z_hbm, r_hbm, out_hbm, cx, cy, cz, rr, c2, ostage):
    wid = lax.axis_index("s") * _NC + lax.axis_index("c")
    b = wid // _CPB
    base_row = _TCR + (wid % _CPB) * _RPW
    pltpu.sync_copy(x_hbm.at[b], cx)
    pltpu.sync_copy(y_hbm.at[b], cy)
    pltpu.sync_copy(z_hbm.at[b], cz)
    pltpu.sync_copy(r_hbm.at[b], rr)

    lane = lax.iota(jnp.int32, 16)
    inf = jnp.float32(jnp.inf)
    z16 = jnp.zeros((16,), jnp.float32)

    def c2_step(c, carry):
        off = c * 16
        xv = cx[pl.ds(off, 16)]
        yv = cy[pl.ds(off, 16)]
        zv = cz[pl.ds(off, 16)]
        c2[pl.ds(off, 16)] = xv * xv + yv * yv + zv * zv
        return carry

    lax.fori_loop(0, _NCHUNK, c2_step, jnp.int32(0))

    def row_group16(t, carry):
        s_acc, s2_acc = carry
        g16 = base_row + t * 16
        rx16 = cx[pl.ds(g16, 16)]
        ry16 = cy[pl.ds(g16, 16)]
        rz16 = cz[pl.ds(g16, 16)]
        rr16 = rr[pl.ds(g16, 16)]
        topv = z16
        for sub in range(16 // _IL):
            idxs = [sub * _IL + j for j in range(_IL)]
            sx = [jnp.full((16,), rx16[i]) for i in idxs]
            sy = [jnp.full((16,), ry16[i]) for i in idxs]
            sz = [jnp.full((16,), rz16[i]) for i in idxs]
            nx = [jnp.float32(-2.0) * v for v in sx]
            ny = [jnp.float32(-2.0) * v for v in sy]
            nz = [jnp.float32(-2.0) * v for v in sz]
            rn2 = [(a * a + b_ * b_) + c_ * c_ for a, b_, c_ in zip(sx, sy, sz)]

            def chunk_step(c, ks):
                off = c * 16
                xv = cx[pl.ds(off, 16)]
                yv = cy[pl.ds(off, 16)]
                zv = cz[pl.ds(off, 16)]
                rv = rr[pl.ds(off, 16)]
                c2v = c2[pl.ds(off, 16)]
                out = []
                for j in range(_IL):
                    kk, kv = ks[2 * j], ks[2 * j + 1]
                    key = ((c2v + xv * nx[j]) + yv * ny[j]) + zv * nz[j]
                    sk, sv = plsc.sort_key_val(key, rv)
                    sel = kk <= sk
                    lok = jnp.where(sel, kk, sk)
                    lov = jnp.where(sel, kv, sv)
                    kk, kv = plsc.sort_key_val(lok, lov, descending=True)
                    out += [kk, kv]
                return tuple(out)

            k0 = (jnp.full((16,), inf), z16) * _IL
            ks = lax.fori_loop(0, _NCHUNK, chunk_step, k0, unroll=4)
            for j in range(_IL):
                d2c = jnp.maximum(ks[2 * j] + rn2[j], jnp.float32(0.0))
                g = _sqrt16(d2c) - ks[2 * j + 1]
                g = jnp.where((lane >= 16 - 1 - _K) & (lane < 15), g, inf)
                top = jnp.min(g) - rr16[idxs[j]]
                topv = jnp.where(lane == idxs[j], jnp.full((16,), top), topv)
        return s_acc + topv, s2_acc + topv * topv

    sv, s2v = lax.fori_loop(0, _RPW // 16, row_group16, (z16, z16))
    s = jnp.sum(sv)
    s2 = jnp.sum(s2v)
    ostage[...] = jnp.where(lane == 0, s, jnp.where(lane == 1, s2, jnp.float32(0.0)))
    pltpu.sync_copy(ostage, out_hbm.at[wid])


def _tc_body(coords_ref, out_ref, d2_ref, acc_ref):
    b = pl.program_id(0)
    j = pl.program_id(1)

    cx = coords_ref[0, 0, :]
    cy = coords_ref[0, 1, :]
    cz = coords_ref[0, 2, :]
    rr = coords_ref[0, 3, :]

    rx = coords_ref[0, 0, pl.ds(j * _R, _R)]
    ry = coords_ref[0, 1, pl.ds(j * _R, _R)]
    rz = coords_ref[0, 2, pl.ds(j * _R, _R)]
    r_row = coords_ref[0, 3, pl.ds(j * _R, _R)]

    dx = rx[:, None] - cx[None, :]
    dy = ry[:, None] - cy[None, :]
    dz = rz[:, None] - cz[None, :]
    d2 = dx * dx + dy * dy + dz * dz

    col = jax.lax.broadcasted_iota(jnp.int32, (_R, _N), 1)
    row = j * _R + jax.lax.broadcasted_iota(jnp.int32, (_R, _N), 0)
    d2 = jnp.where(col == row, jnp.inf, d2)
    d2_ref[...] = d2

    inf = jnp.float32(jnp.inf)
    _CB = 256
    _NB = _N // _CB

    def round_fn(_, carry):
        thr, ans = carry
        bms = []
        brs = []
        for blk in range(_NB):
            sl = d2_ref[:, blk * _CB:(blk + 1) * _CB]
            rb = rr[None, blk * _CB:(blk + 1) * _CB]
            masked = jnp.where(sl > thr[:, None], sl, inf)
            bm = jnp.min(masked, axis=1)
            braw = jnp.max(jnp.where(masked == bm[:, None], rb, -inf), axis=1)
            bms.append(bm)
            brs.append(jnp.where(bm < inf, braw, -inf))
        m = bms[0]
        for blk in range(1, _NB):
            m = jnp.minimum(m, bms[blk])
        rbest = jnp.full((_R,), -inf)
        for blk in range(_NB):
            rbest = jnp.maximum(rbest, jnp.where(bms[blk] == m, brs[blk], -inf))
        ans = jnp.minimum(ans, jnp.sqrt(m) - rbest)
        return m, ans

    thr0 = jnp.full((_R,), -inf)
    ans0 = jnp.full((_R,), inf)
    _, ans = jax.lax.fori_loop(0, _K, round_fn, (thr0, ans0))

    top = ans - r_row
    s = jnp.sum(top)
    s2 = jnp.sum(top * top)

    @pl.when(j == 0)
    def _():
        acc_ref[0] = s
        acc_ref[1] = s2

    @pl.when(j > 0)
    def _():
        acc_ref[0] = acc_ref[0] + s
        acc_ref[1] = acc_ref[1] + s2

    @pl.when(j == _JT - 1)
    def _():
        out_ref[pl.ds(2 * b, 1), :] = jnp.full((1, 128), acc_ref[0])
        out_ref[pl.ds(2 * b + 1, 1), :] = jnp.full((1, 128), acc_ref[1])


def _tc_part(coords):
    return pl.pallas_call(
        _tc_body,
        grid=(_B, _JT),
        in_specs=[pl.BlockSpec((1, 4, _N), lambda b, j: (b, 0, 0))],
        out_specs=pl.BlockSpec((8, 128), lambda b, j: (0, 0)),
        out_shape=jax.ShapeDtypeStruct((8, 128), jnp.float32),
        scratch_shapes=[
            pltpu.VMEM((_R, _N), jnp.float32),
            pltpu.SMEM((2,), jnp.float32),
        ],
    )(coords)


def _fin_body(p_ref, q_ref, out_ref):
    p = p_ref[...]
    q = q_ref[...]
    ri = lax.broadcasted_iota(jnp.int32, (_NW, 16), 0)
    ci = lax.broadcasted_iota(jnp.int32, (_NW, 16), 1)
    qi = lax.broadcasted_iota(jnp.int32, (8, 128), 0)
    qc = lax.broadcasted_iota(jnp.int32, (8, 128), 1)
    n = jnp.float32(_N)
    tot = jnp.float32(0.0)
    for b in range(_B):
        in_b = ri // _CPB == b
        s = jnp.sum(jnp.where(in_b & (ci == 0), p, 0.0))
        s2 = jnp.sum(jnp.where(in_b & (ci == 1), p, 0.0))
        s = s + jnp.sum(jnp.where((qi == 2 * b) & (qc == 0), q, 0.0))
        s2 = s2 + jnp.sum(jnp.where((qi == 2 * b + 1) & (qc == 0), q, 0.0))
        var = (s2 - s * s / n) / (n - 1.0)
        tot = tot + var
    out_ref[...] = jnp.full((8, 128), tot / jnp.float32(_B))


def _finalize(partials_sc, partials_tc):
    out = pl.pallas_call(
        _fin_body,
        out_shape=jax.ShapeDtypeStruct((8, 128), jnp.float32),
    )(partials_sc, partials_tc)
    return out[0, 0]


@jax.jit
def kernel(spheres):
    coords = jnp.transpose(spheres, (0, 2, 1))
    cx = coords[:, 0]
    cy = coords[:, 1]
    cz = coords[:, 2]
    rr = coords[:, 3]
    partials_sc = _sc_topk(cx, cy, cz, rr)
    partials_tc = _tc_part(coords)
    return _finalize(partials_sc, partials_tc)


# speedup vs baseline: 1.4706x; 1.4706x over previous
import functools

import jax
import jax.numpy as jnp
from jax import lax
from jax.experimental import pallas as pl
from jax.experimental.pallas import tpu as pltpu
from jax.experimental.pallas import tpu_sc as plsc

_B = 4
_N = 2048
_K = 10

_TCR = 768
_R = 768
_JT = _TCR // _R

_NC = 2
_NS = 16
_NW = _NC * _NS
_CPB = _NW // _B
_SCR = _N - _TCR
_RPW = _SCR // _CPB
_IL = 4
_NCHUNK = _N // 16


def _sqrt16(x):
    i = lax.bitcast_convert_type(x, jnp.int32)
    y = lax.bitcast_convert_type(jnp.int32(0x5F3759DF) - (i >> 1), jnp.float32)
    for _ in range(3):
        y = y * (jnp.float32(1.5) - jnp.float32(0.5) * x * y * y)
    return x * y


_mesh = plsc.VectorSubcoreMesh(
    core_axis_name="c", subcore_axis_name="s", num_cores=_NC, num_subcores=_NS
)


@functools.partial(
    pl.kernel,
    out_type=jax.ShapeDtypeStruct((_NW, 16), jnp.float32),
    mesh=_mesh,
    compiler_params=pltpu.CompilerParams(needs_layout_passes=False),
    scratch_types=[
        pltpu.VMEM((_N,), jnp.float32),
        pltpu.VMEM((_N,), jnp.float32),
        pltpu.VMEM((_N,), jnp.float32),
        pltpu.VMEM((_N,), jnp.float32),
        pltpu.VMEM((_N,), jnp.float32),
        pltpu.VMEM((16,), jnp.float32),
    ],
)
def _sc_topk(coords_hbm, out_hbm, cx, cy, cz, rr, c2, ostage):
    wid = lax.axis_index("s") * _NC + lax.axis_index("c")
    b = wid // _CPB
    base_row = _TCR + (wid % _CPB) * _RPW
    pltpu.sync_copy(coords_hbm.at[b, 0], cx)
    pltpu.sync_copy(coords_hbm.at[b, 1], cy)
    pltpu.sync_copy(coords_hbm.at[b, 2], cz)
    pltpu.sync_copy(coords_hbm.at[b, 3], rr)

    lane = lax.iota(jnp.int32, 16)
    inf = jnp.float32(jnp.inf)
    z16 = jnp.zeros((16,), jnp.float32)

    def c2_step(c, carry):
        off = c * 16
        xv = cx[pl.ds(off, 16)]
        yv = cy[pl.ds(off, 16)]
        zv = cz[pl.ds(off, 16)]
        c2[pl.ds(off, 16)] = xv * xv + yv * yv + zv * zv
        return carry

    lax.fori_loop(0, _NCHUNK, c2_step, jnp.int32(0))

    def row_group16(t, carry):
        s_acc, s2_acc = carry
        g16 = base_row + t * 16
        rx16 = cx[pl.ds(g16, 16)]
        ry16 = cy[pl.ds(g16, 16)]
        rz16 = cz[pl.ds(g16, 16)]
        rr16 = rr[pl.ds(g16, 16)]
        topv = z16
        for sub in range(16 // _IL):
            idxs = [sub * _IL + j for j in range(_IL)]
            sx = [jnp.full((16,), rx16[i]) for i in idxs]
            sy = [jnp.full((16,), ry16[i]) for i in idxs]
            sz = [jnp.full((16,), rz16[i]) for i in idxs]
            nx = [jnp.float32(-2.0) * v for v in sx]
            ny = [jnp.float32(-2.0) * v for v in sy]
            nz = [jnp.float32(-2.0) * v for v in sz]
            rn2 = [(a * a + b_ * b_) + c_ * c_ for a, b_, c_ in zip(sx, sy, sz)]

            def chunk_step(c, ks):
                off = c * 16
                xv = cx[pl.ds(off, 16)]
                yv = cy[pl.ds(off, 16)]
                zv = cz[pl.ds(off, 16)]
                rv = rr[pl.ds(off, 16)]
                c2v = c2[pl.ds(off, 16)]
                out = []
                for j in range(_IL):
                    kk, kv = ks[2 * j], ks[2 * j + 1]
                    key = ((c2v + xv * nx[j]) + yv * ny[j]) + zv * nz[j]
                    sk, sv = plsc.sort_key_val(key, rv)
                    sel = kk <= sk
                    lok = jnp.where(sel, kk, sk)
                    lov = jnp.where(sel, kv, sv)
                    kk, kv = plsc.sort_key_val(lok, lov, descending=True)
                    out += [kk, kv]
                return tuple(out)

            k0 = (jnp.full((16,), inf), z16) * _IL
            ks = lax.fori_loop(0, _NCHUNK, chunk_step, k0, unroll=4)
            for j in range(_IL):
                d2c = jnp.maximum(ks[2 * j] + rn2[j], jnp.float32(0.0))
                g = _sqrt16(d2c) - ks[2 * j + 1]
                g = jnp.where((lane >= 16 - 1 - _K) & (lane < 15), g, inf)
                top = jnp.min(g) - rr16[idxs[j]]
                topv = jnp.where(lane == idxs[j], jnp.full((16,), top), topv)
        return s_acc + topv, s2_acc + topv * topv

    sv, s2v = lax.fori_loop(0, _RPW // 16, row_group16, (z16, z16))
    s = jnp.sum(sv)
    s2 = jnp.sum(s2v)
    ostage[...] = jnp.where(lane == 0, s, jnp.where(lane == 1, s2, jnp.float32(0.0)))
    pltpu.sync_copy(ostage, out_hbm.at[wid])


def _tc_body(coords_ref, out_ref, d2_ref, acc_ref):
    b = pl.program_id(0)
    j = pl.program_id(1)

    cx = coords_ref[0, 0, :]
    cy = coords_ref[0, 1, :]
    cz = coords_ref[0, 2, :]
    rr = coords_ref[0, 3, :]

    rx = coords_ref[0, 0, pl.ds(j * _R, _R)]
    ry = coords_ref[0, 1, pl.ds(j * _R, _R)]
    rz = coords_ref[0, 2, pl.ds(j * _R, _R)]
    r_row = coords_ref[0, 3, pl.ds(j * _R, _R)]

    dx = rx[:, None] - cx[None, :]
    dy = ry[:, None] - cy[None, :]
    dz = rz[:, None] - cz[None, :]
    d2 = dx * dx + dy * dy + dz * dz

    col = jax.lax.broadcasted_iota(jnp.int32, (_R, _N), 1)
    row = j * _R + jax.lax.broadcasted_iota(jnp.int32, (_R, _N), 0)
    d2 = jnp.where(col == row, jnp.inf, d2)
    d2_ref[...] = d2

    r_col = rr[None, :]
    inf = jnp.float32(jnp.inf)

    def round_fn(_, carry):
        thr, ans = carry
        dv = d2_ref[...]
        m = jnp.min(jnp.where(dv > thr[:, None], dv, inf), axis=1)
        rbest = jnp.max(jnp.where(dv == m[:, None], r_col, -inf), axis=1)
        ans = jnp.minimum(ans, jnp.sqrt(m) - rbest)
        return m, ans

    thr0 = jnp.full((_R,), -inf)
    ans0 = jnp.full((_R,), inf)
    _, ans = jax.lax.fori_loop(0, _K, round_fn, (thr0, ans0))

    top = ans - r_row
    s = jnp.sum(top)
    s2 = jnp.sum(top * top)

    @pl.when(j == 0)
    def _():
        acc_ref[0] = s
        acc_ref[1] = s2

    @pl.when(j > 0)
    def _():
        acc_ref[0] = acc_ref[0] + s
        acc_ref[1] = acc_ref[1] + s2

    @pl.when(j == _JT - 1)
    def _():
        out_ref[pl.ds(2 * b, 1), :] = jnp.full((1, 128), acc_ref[0])
        out_ref[pl.ds(2 * b + 1, 1), :] = jnp.full((1, 128), acc_ref[1])


def _tc_part(coords):
    return pl.pallas_call(
        _tc_body,
        grid=(_B, _JT),
        in_specs=[pl.BlockSpec((1, 4, _N), lambda b, j: (b, 0, 0))],
        out_specs=pl.BlockSpec((8, 128), lambda b, j: (0, 0)),
        out_shape=jax.ShapeDtypeStruct((8, 128), jnp.float32),
        scratch_shapes=[
            pltpu.VMEM((_R, _N), jnp.float32),
            pltpu.SMEM((2,), jnp.float32),
        ],
    )(coords)


def _fin_body(p_ref, q_ref, out_ref):
    p = p_ref[...]
    q = q_ref[...]
    ri = lax.broadcasted_iota(jnp.int32, (_NW, 16), 0)
    ci = lax.broadcasted_iota(jnp.int32, (_NW, 16), 1)
    qi = lax.broadcasted_iota(jnp.int32, (8, 128), 0)
    qc = lax.broadcasted_iota(jnp.int32, (8, 128), 1)
    n = jnp.float32(_N)
    tot = jnp.float32(0.0)
    for b in range(_B):
        in_b = ri // _CPB == b
        s = jnp.sum(jnp.where(in_b & (ci == 0), p, 0.0))
        s2 = jnp.sum(jnp.where(in_b & (ci == 1), p, 0.0))
        s = s + jnp.sum(jnp.where((qi == 2 * b) & (qc == 0), q, 0.0))
        s2 = s2 + jnp.sum(jnp.where((qi == 2 * b + 1) & (qc == 0), q, 0.0))
        var = (s2 - s * s / n) / (n - 1.0)
        tot = tot + var
    out_ref[...] = jnp.full((8, 128), tot / jnp.float32(_B))


def _finalize(partials_sc, partials_tc):
    out = pl.pallas_call(
        _fin_body,
        out_shape=jax.ShapeDtypeStruct((8, 128), jnp.float32),
    )(partials_sc, partials_tc)
    return out[0, 0]


@jax.jit
def kernel(spheres):
    coords = jnp.transpose(spheres, (0, 2, 1))
    partials_sc = _sc_topk(coords)
    partials_tc = _tc_part(coords)
    return _finalize(partials_sc, partials_tc)
